# baseline (speedup 1.0000x reference)
import jax
import jax.numpy as jnp
from jax.experimental import pallas as pl
from jax.experimental.pallas import tpu as pltpu


def _body(l_ref, f_ref, fm_ref, w1t_ref, w2t_ref, b_ref, out_ref):
    x = jnp.dot(
        l_ref[...].astype(jnp.bfloat16),
        f_ref[...],
        preferred_element_type=jnp.float32,
    )
    f = fm_ref[...]
    out_ref[...] = (
        jnp.dot(f + x, w1t_ref[...], preferred_element_type=jnp.float32)
        + jnp.dot(x * f, w2t_ref[...], preferred_element_type=jnp.float32)
        + b_ref[...]
    )


def kernel(lap_matrix, eye_matrix, features, W1, b1, W2, b2):
    n, d = features.shape
    bm = min(384, n)
    nm = pl.cdiv(n, bm)

    bias = (b1 + b2).reshape(1, d)
    f_bf16 = features.astype(jnp.bfloat16)

    in_specs = [
        pl.BlockSpec((bm, n), lambda i: (i, 0)),
        pl.BlockSpec((n, d), lambda i: (0, 0)),
        pl.BlockSpec((bm, d), lambda i: (i, 0)),
        pl.BlockSpec((d, d), lambda i: (0, 0)),
        pl.BlockSpec((d, d), lambda i: (0, 0)),
        pl.BlockSpec((1, d), lambda i: (0, 0)),
    ]

    return pl.pallas_call(
        _body,
        grid=(nm,),
        in_specs=in_specs,
        out_specs=pl.BlockSpec((bm, d), lambda i: (i, 0)),
        out_shape=jax.ShapeDtypeStruct((n, d), jnp.float32),
        compiler_params=pltpu.CompilerParams(
            dimension_semantics=("arbitrary",),
            vmem_limit_bytes=63 * 1024 * 1024,
        ),
    )(lap_matrix, f_bf16, features, W1.T, W2.T, bias)
